# baseline (speedup 1.0000x reference)
import functools

import jax
import jax.numpy as jnp
from jax import lax
from jax.experimental import pallas as pl
from jax.experimental.pallas import tpu as pltpu
from jax.experimental.pallas import tpu_sc as plsc

_NC = 2
_NS = 16
_L = 16
_NW = _NC * _NS


@functools.partial(jax.jit, static_argnames=("n", "e", "d"))
def _midpoints(data2, eidx, *, n, e, d):
    rows_per_w = d // _NW
    mesh = plsc.VectorSubcoreMesh(core_axis_name="c", subcore_axis_name="s")

    @functools.partial(
        pl.kernel,
        out_type=jax.ShapeDtypeStruct((d, n + e), jnp.float32),
        mesh=mesh,
        scratch_types=[
            pltpu.VMEM((e,), jnp.int32),
            pltpu.VMEM((n,), jnp.float32),
            pltpu.VMEM((n,), jnp.float32),
            pltpu.VMEM((e,), jnp.float32),
            pltpu.VMEM((e,), jnp.float32),
            pltpu.VMEM((n,), jnp.float32),
            pltpu.SemaphoreType.DMA,
            pltpu.SemaphoreType.DMA,
            pltpu.SemaphoreType.DMA,
            pltpu.SemaphoreType.DMA,
            pltpu.SemaphoreType.DMA,
        ],
        compiler_params=pltpu.CompilerParams(needs_layout_passes=False),
    )
    def k(data_hbm, eidx_hbm, out_hbm,
          idx_v, row0_v, row1_v, mid0_v, mid1_v, zero_v,
          sem_in0, sem_in1, sem_mid0, sem_mid1, sem_zero):
        wid = lax.axis_index("s") * _NC + lax.axis_index("c")
        base = wid * rows_per_w
        rows = (row0_v, row1_v)
        mids = (mid0_v, mid1_v)
        sems_in = (sem_in0, sem_in1)
        sems_mid = (sem_mid0, sem_mid1)

        pltpu.async_copy(data_hbm.at[base], row0_v, sem_in0)
        pltpu.async_copy(data_hbm.at[base + 1], row1_v, sem_in1)
        idx_cp = pltpu.async_copy(eidx_hbm, idx_v, sem_zero)

        @plsc.parallel_loop(0, n // _L, 1, unroll=8)
        def _(i):
            zero_v[pl.ds(i * _L, _L)] = jnp.zeros((_L,), jnp.float32)

        idx_cp.wait()

        for rr in (0, 1):
            pltpu.async_copy(zero_v, out_hbm.at[base + rr, pl.ds(0, n)],
                             sem_zero)

        @pl.loop(0, rows_per_w, step=2)
        def _(r):
            for b in (0, 1):
                ridx = base + r + b
                row_v = rows[b]
                mid_v = mids[b]
                pltpu.make_async_copy(data_hbm.at[ridx], row_v,
                                      sems_in[b]).wait()

                @pl.when(r >= 2)
                def _():
                    pltpu.make_async_copy(
                        mid_v, out_hbm.at[ridx - 2, pl.ds(n, e)],
                        sems_mid[b]).wait()

                @plsc.parallel_loop(0, e // _L, 1, unroll=8)
                def _(j):
                    s = j * _L
                    p = idx_v[pl.ds(s, _L)]
                    i0 = jnp.bitwise_and(p, 0xFFFF)
                    i1 = lax.shift_right_logical(p, 16)
                    v0 = plsc.load_gather(row_v, [i0])
                    v1 = plsc.load_gather(row_v, [i1])
                    mid_v[pl.ds(s, _L)] = (v0 + v1) * 0.5

                @pl.when(r + 2 < rows_per_w)
                def _():
                    pltpu.async_copy(data_hbm.at[ridx + 2], row_v,
                                     sems_in[b])

                pltpu.async_copy(mid_v, out_hbm.at[ridx, pl.ds(n, e)],
                                 sems_mid[b])

                @pl.when(r >= 2)
                def _():
                    pltpu.async_copy(zero_v, out_hbm.at[ridx, pl.ds(0, n)],
                                     sem_zero)

        for b in (0, 1):
            pltpu.make_async_copy(
                mids[b], out_hbm.at[base + rows_per_w - 2 + b, pl.ds(n, e)],
                sems_mid[b]).wait()

        @pl.loop(0, rows_per_w)
        def _(r):
            pltpu.make_async_copy(zero_v, out_hbm.at[base + r, pl.ds(0, n)],
                                  sem_zero).wait()

    return k(data2, eidx)


def kernel(data, edges):
    b, c, n = data.shape
    e = edges.shape[0]
    d = b * c
    data2 = data.reshape(d, n)
    eidx = edges[:, 0] | (edges[:, 1] << 16)
    out2 = _midpoints(data2, eidx, n=n, e=e, d=d)
    return out2.reshape(b, c, n + e)


# speedup vs baseline: 1.0057x; 1.0057x over previous
import functools

import jax
import jax.numpy as jnp
from jax import lax
from jax.experimental import pallas as pl
from jax.experimental.pallas import tpu as pltpu
from jax.experimental.pallas import tpu_sc as plsc

_NC = 2
_NS = 16
_L = 16
_NW = _NC * _NS


@functools.partial(jax.jit, static_argnames=("n", "e", "d"))
def _midpoints(data2, eidx, *, n, e, d):
    rows_per_w = d // _NW
    mesh = plsc.VectorSubcoreMesh(core_axis_name="c", subcore_axis_name="s")

    @functools.partial(
        pl.kernel,
        out_type=jax.ShapeDtypeStruct((d, n + e), jnp.float32),
        mesh=mesh,
        scratch_types=[
            pltpu.VMEM((e,), jnp.int32),
            pltpu.VMEM((n,), jnp.float32),
            pltpu.VMEM((n,), jnp.float32),
            pltpu.VMEM((e,), jnp.float32),
            pltpu.VMEM((e,), jnp.float32),
            pltpu.VMEM((n,), jnp.float32),
            pltpu.SemaphoreType.DMA,
            pltpu.SemaphoreType.DMA,
            pltpu.SemaphoreType.DMA,
            pltpu.SemaphoreType.DMA,
            pltpu.SemaphoreType.DMA,
        ],
        compiler_params=pltpu.CompilerParams(needs_layout_passes=False),
    )
    def k(data_hbm, eidx_hbm, out_hbm,
          idx_v, row0_v, row1_v, mid0_v, mid1_v, zero_v,
          sem_in0, sem_in1, sem_mid0, sem_mid1, sem_zero):
        wid = lax.axis_index("s") * _NC + lax.axis_index("c")
        base = wid * rows_per_w
        rows = (row0_v, row1_v)
        mids = (mid0_v, mid1_v)
        sems_in = (sem_in0, sem_in1)
        sems_mid = (sem_mid0, sem_mid1)

        pltpu.async_copy(data_hbm.at[base], row0_v, sem_in0)
        pltpu.async_copy(data_hbm.at[base + 1], row1_v, sem_in1)
        idx_cp = pltpu.async_copy(eidx_hbm, idx_v, sem_zero)

        @plsc.parallel_loop(0, n // _L, 1, unroll=8)
        def _(i):
            zero_v[pl.ds(i * _L, _L)] = jnp.zeros((_L,), jnp.float32)

        idx_cp.wait()

        @pl.loop(0, rows_per_w, step=2)
        def _(r):
            for b in (0, 1):
                ridx = base + r + b
                row_v = rows[b]
                mid_v = mids[b]
                pltpu.make_async_copy(data_hbm.at[ridx], row_v,
                                      sems_in[b]).wait()

                @pl.when(r >= 2)
                def _():
                    pltpu.make_async_copy(
                        mid_v, out_hbm.at[ridx - 2, pl.ds(n, e)],
                        sems_mid[b]).wait()

                @plsc.parallel_loop(0, e // _L, 1, unroll=8)
                def _(j):
                    s = j * _L
                    p = idx_v[pl.ds(s, _L)]
                    i0 = jnp.bitwise_and(p, 0xFFFF)
                    i1 = lax.shift_right_logical(p, 16)
                    v0 = plsc.load_gather(row_v, [i0])
                    v1 = plsc.load_gather(row_v, [i1])
                    mid_v[pl.ds(s, _L)] = (v0 + v1) * 0.5

                @pl.when(r + 2 < rows_per_w)
                def _():
                    pltpu.async_copy(data_hbm.at[ridx + 2], row_v,
                                     sems_in[b])

                pltpu.async_copy(mid_v, out_hbm.at[ridx, pl.ds(n, e)],
                                 sems_mid[b])
                pltpu.async_copy(zero_v, out_hbm.at[ridx, pl.ds(0, n)],
                                 sem_zero)

        for b in (0, 1):
            pltpu.make_async_copy(
                mids[b], out_hbm.at[base + rows_per_w - 2 + b, pl.ds(n, e)],
                sems_mid[b]).wait()

        @pl.loop(0, rows_per_w)
        def _(r):
            pltpu.make_async_copy(zero_v, out_hbm.at[base + r, pl.ds(0, n)],
                                  sem_zero).wait()

    return k(data2, eidx)


def kernel(data, edges):
    b, c, n = data.shape
    e = edges.shape[0]
    d = b * c
    data2 = data.reshape(d, n)
    eidx = edges[:, 0] | (edges[:, 1] << 16)
    out2 = _midpoints(data2, eidx, n=n, e=e, d=d)
    return out2.reshape(b, c, n + e)
